# Initial kernel scaffold; baseline (speedup 1.0000x reference)
import jax
import jax.numpy as jnp
from jax.experimental import pallas as pl


def kernel(x, edge_index, Ws0, Wd0, al0, ar0, b0, Ws1, Wd1, al1, ar1, b1, Ws2, Wd2, al2, ar2, b2):
    raise NotImplementedError("write your pallas kernel here")


# speedup vs baseline: 1.0570x; 1.0570x over previous
import jax
import jax.numpy as jnp
from jax.experimental import pallas as pl

N = 10000
H = 8
DH = 8
C = 40


def _gat_layer(h, src, dst, Ws, Wd, al, ar, b, act):
    n = h.shape[0]
    Hn, dout = al.shape
    fs = (h @ Ws).reshape(n, Hn, dout)
    fd = (h @ Wd).reshape(n, Hn, dout)
    el = (fs * al[None, :, :]).sum(-1)
    er = (fd * ar[None, :, :]).sum(-1)
    e = jax.nn.leaky_relu(el[src] + er[dst], 0.2)
    ee = jnp.exp(e)
    den = jax.ops.segment_sum(ee, dst, num_segments=n)
    num = jax.ops.segment_sum(ee[:, :, None] * fs[src], dst, num_segments=n)
    out = num / jnp.maximum(den, 1e-9)[:, :, None]
    out = out + b.reshape(1, Hn, dout)
    if act:
        out = jax.nn.relu(out)
    return out


def _logsoftmax_kernel(x_ref, o_ref):
    x = x_ref[...]
    m = jnp.max(x, axis=-1, keepdims=True)
    s = x - m
    lse = jnp.log(jnp.sum(jnp.exp(s), axis=-1, keepdims=True))
    o_ref[...] = s - lse


def kernel(x, edge_index, Ws0, Wd0, al0, ar0, b0, Ws1, Wd1, al1, ar1, b1, Ws2, Wd2, al2, ar2, b2):
    src = edge_index[0]
    dst = edge_index[1]
    h = _gat_layer(x, src, dst, Ws0, Wd0, al0, ar0, b0, True).reshape(N, H * DH)
    h = _gat_layer(h, src, dst, Ws1, Wd1, al1, ar1, b1, True).reshape(N, H * DH)
    h = _gat_layer(h, src, dst, Ws2, Wd2, al2, ar2, b2, False)
    h = h.mean(axis=1)
    return pl.pallas_call(
        _logsoftmax_kernel,
        out_shape=jax.ShapeDtypeStruct((N, C), jnp.float32),
    )(h)


# speedup vs baseline: 34.4331x; 32.5752x over previous
import functools

import jax
import jax.numpy as jnp
import numpy as np
from jax import lax
from jax.experimental import pallas as pl
from jax.experimental.pallas import tpu as pltpu
from jax.experimental.pallas import tpu_sc as plsc

N = 10000
E = 320000
D = 128
H = 8
DH = 8
C = 40

_BM = 1000
_K0 = 80


def _tc_prep0(x, wcat):
    def body(x_ref, w_ref, a_ref, e_ref):
        p = jnp.dot(x_ref[...], w_ref[...], preferred_element_type=jnp.float32)
        z56 = jnp.zeros((_BM, 56), jnp.float32)
        z120 = jnp.zeros((_BM, 120), jnp.float32)
        a_ref[...] = jnp.concatenate([p[:, 0:72], z56], axis=1)
        e_ref[...] = jnp.concatenate([p[:, 72:80], z120], axis=1)

    return pl.pallas_call(
        body,
        grid=(N // _BM,),
        in_specs=[
            pl.BlockSpec((_BM, D), lambda i: (i, 0)),
            pl.BlockSpec((D, 80), lambda i: (0, 0)),
        ],
        out_specs=[
            pl.BlockSpec((_BM, 128), lambda i: (i, 0)),
            pl.BlockSpec((_BM, 128), lambda i: (i, 0)),
        ],
        out_shape=[
            jax.ShapeDtypeStruct((N, 128), jnp.float32),
            jax.ShapeDtypeStruct((N, 128), jnp.float32),
        ],
    )(x, wcat)


def _combine_h(acc_ref, b_ref, b8_ref):
    a0 = acc_ref[0]
    a1 = acc_ref[1]
    num = a0[:, 0:64] + a1[:, 0:64]
    den = a0[:, 64:72] + a1[:, 64:72]
    r = 1.0 / jnp.maximum(den, 1e-9)
    rb = jnp.dot(r, b8_ref[...], preferred_element_type=jnp.float32)
    return jnp.maximum(num * rb + b_ref[...], 0.0)


def _tc_mid(acc, b, wcat, b8):
    def body(acc_ref, b_ref, w_ref, b8_ref, a_ref, e_ref):
        h = _combine_h(acc_ref, b_ref, b8_ref)
        p = jnp.dot(h, w_ref[...], preferred_element_type=jnp.float32)
        z56 = jnp.zeros((_BM, 56), jnp.float32)
        z120 = jnp.zeros((_BM, 120), jnp.float32)
        a_ref[...] = jnp.concatenate([p[:, 0:72], z56], axis=1)
        e_ref[...] = jnp.concatenate([p[:, 72:80], z120], axis=1)

    return pl.pallas_call(
        body,
        grid=(N // _BM,),
        in_specs=[
            pl.BlockSpec((2, _BM, 128), lambda i: (0, i, 0)),
            pl.BlockSpec((1, 64), lambda i: (0, 0)),
            pl.BlockSpec((64, 80), lambda i: (0, 0)),
            pl.BlockSpec((8, 64), lambda i: (0, 0)),
        ],
        out_specs=[
            pl.BlockSpec((_BM, 128), lambda i: (i, 0)),
            pl.BlockSpec((_BM, 128), lambda i: (i, 0)),
        ],
        out_shape=[
            jax.ShapeDtypeStruct((N, 128), jnp.float32),
            jax.ShapeDtypeStruct((N, 128), jnp.float32),
        ],
    )(acc, b, wcat, b8)


def _tc_prep2(acc, b, wcat, b8):
    def body(acc_ref, b_ref, w_ref, b8_ref, aa_ref, ab_ref, ea_ref, eb_ref):
        h = _combine_h(acc_ref, b_ref, b8_ref)
        p = jnp.dot(h, w_ref[...], preferred_element_type=jnp.float32)
        fs = p[:, 0:320]
        el = p[:, 320:328]
        er = p[:, 328:336]
        z126 = jnp.zeros((_BM, 126), jnp.float32)
        z46 = jnp.zeros((_BM, 46), jnp.float32)
        for g, (a_ref, e_ref) in enumerate(
                [(aa_ref, ea_ref)] * 2 + [(ab_ref, eb_ref)] * 2):
            j = g % 2
            a_ref[j] = jnp.concatenate(
                [fs[:, 80 * g:80 * g + 80], el[:, 2 * g:2 * g + 2], z46],
                axis=1)
            e_ref[j] = jnp.concatenate([er[:, 2 * g:2 * g + 2], z126], axis=1)

    return pl.pallas_call(
        body,
        grid=(N // _BM,),
        in_specs=[
            pl.BlockSpec((2, _BM, 128), lambda i: (0, i, 0)),
            pl.BlockSpec((1, 64), lambda i: (0, 0)),
            pl.BlockSpec((64, 336), lambda i: (0, 0)),
            pl.BlockSpec((8, 64), lambda i: (0, 0)),
        ],
        out_specs=[
            pl.BlockSpec((2, _BM, 128), lambda i: (0, i, 0)),
            pl.BlockSpec((2, _BM, 128), lambda i: (0, i, 0)),
            pl.BlockSpec((2, _BM, 128), lambda i: (0, i, 0)),
            pl.BlockSpec((2, _BM, 128), lambda i: (0, i, 0)),
        ],
        out_shape=[
            jax.ShapeDtypeStruct((2, N, 128), jnp.float32),
            jax.ShapeDtypeStruct((2, N, 128), jnp.float32),
            jax.ShapeDtypeStruct((2, N, 128), jnp.float32),
            jax.ShapeDtypeStruct((2, N, 128), jnp.float32),
        ],
    )(acc, b, wcat, b8)


def _tc_final(acca, accb, bc2, s80, bbar):
    def body(aa_ref, ab_ref, bc2_ref, s80_ref, bb_ref, o_ref):
        t = None
        for ref in (aa_ref, ab_ref):
            for j in (0, 1):
                a = ref[j]
                num = a[:, 0:80]
                den = a[:, 80:82]
                r = 1.0 / jnp.maximum(den, 1e-9)
                rb = jnp.dot(r, bc2_ref[...],
                             preferred_element_type=jnp.float32)
                u = jnp.dot(num * rb, s80_ref[...],
                            preferred_element_type=jnp.float32)
                t = u if t is None else t + u
        t = t * (1.0 / H) + bb_ref[...]
        m = jnp.max(t, axis=1, keepdims=True)
        s = t - m
        o_ref[...] = s - jnp.log(jnp.sum(jnp.exp(s), axis=1, keepdims=True))

    return pl.pallas_call(
        body,
        grid=(N // _BM,),
        in_specs=[
            pl.BlockSpec((2, _BM, 128), lambda i: (0, i, 0)),
            pl.BlockSpec((2, _BM, 128), lambda i: (0, i, 0)),
            pl.BlockSpec((2, 80), lambda i: (0, 0)),
            pl.BlockSpec((80, 40), lambda i: (0, 0)),
            pl.BlockSpec((1, 40), lambda i: (0, 0)),
        ],
        out_specs=pl.BlockSpec((_BM, 40), lambda i: (i, 0)),
        out_shape=jax.ShapeDtypeStruct((N, 40), jnp.float32),
    )(acca, accb, bc2, s80, bbar)


def _vbcast(vec, idx):
    dnums = lax.GatherDimensionNumbers(
        offset_dims=(), collapsed_slice_dims=(0,), start_index_map=(0,))
    return lax.gather(vec, idx[:, None], dnums, (1,),
                      mode=lax.GatherScatterMode.PROMISE_IN_BOUNDS)


def _make_sc_edge(l2):
    aoff = 80 if l2 else 64
    per_w = E // (16 if l2 else 32)
    n_chunks = per_w // _K0

    mesh = plsc.VectorSubcoreMesh(core_axis_name="c", subcore_axis_name="s")

    scratch = [
        pltpu.VMEM((_K0,), jnp.int32),
        pltpu.VMEM((_K0,), jnp.int32),
        pltpu.VMEM((_K0,), jnp.int32),
        pltpu.VMEM((_K0, 128), jnp.float32),
        pltpu.VMEM((_K0, 128), jnp.float32),
        pltpu.VMEM_SHARED((N, 128), jnp.float32),
        pltpu.SemaphoreType.DMA,
        pltpu.SemaphoreType.DMA,
    ]

    @functools.partial(
        pl.kernel,
        out_type=jax.ShapeDtypeStruct((2 * N, 128), jnp.float32),
        mesh=mesh,
        scratch_types=scratch,
    )
    def sc_edge(src_h, dst_h, a_h, er_h, z_h, out_h,
                src_v, dst_v, dst2_v, o_v, er_v, acc, sem1, sem2):
        cid = lax.axis_index("c")
        sid = lax.axis_index("s")
        iota = lax.broadcasted_iota(jnp.int32, (16,), 0)
        pat_h = lax.shift_right_logical(iota, 3)
        pat_0 = jnp.bitwise_and(iota, 0)

        @pl.when(sid == 0)
        def _():
            pltpu.sync_copy(z_h, acc)

        plsc.subcore_barrier()

        ebase = (sid if l2 else sid * 2 + cid) * per_w
        toff = cid * N if l2 else 0

        def chunk(i, carry):
            eb = ebase + i * _K0
            pltpu.sync_copy(src_h.at[pl.ds(eb, _K0)], src_v)
            pltpu.sync_copy(dst_h.at[pl.ds(eb, _K0)], dst_v)
            for j in range(_K0 // 16):
                js = pl.ds(16 * j, 16)
                src_v[js] = src_v[js] + toff
                dst2_v[js] = dst_v[js] + toff
            pltpu.async_copy(a_h.at[src_v], o_v, sem1).wait()
            pltpu.async_copy(er_h.at[dst2_v], er_v, sem2).wait()

            def edge(k, c2):
                a = o_v[k, pl.ds(aoff, 16)]
                b = er_v[k, pl.ds(0, 16)]
                s = a + b
                ee = jnp.exp(jnp.maximum(s, 0.2 * s))
                o_v[k, pl.ds(aoff, 16)] = ee
                if l2:
                    bc01 = _vbcast(ee, pat_0)
                    bc2 = _vbcast(ee, pat_h)
                    bc34 = _vbcast(ee, pat_0 + 1)
                    o_v[k, pl.ds(0, 16)] = o_v[k, pl.ds(0, 16)] * bc01
                    o_v[k, pl.ds(16, 16)] = o_v[k, pl.ds(16, 16)] * bc01
                    o_v[k, pl.ds(32, 16)] = o_v[k, pl.ds(32, 16)] * bc2
                    o_v[k, pl.ds(48, 16)] = o_v[k, pl.ds(48, 16)] * bc34
                    o_v[k, pl.ds(64, 16)] = o_v[k, pl.ds(64, 16)] * bc34
                else:
                    for v in range(4):
                        bc = _vbcast(ee, pat_h + 2 * v)
                        vs = pl.ds(16 * v, 16)
                        o_v[k, vs] = o_v[k, vs] * bc
                return c2

            lax.fori_loop(0, _K0, edge, 0)
            pltpu.sync_copy(o_v, acc.at[dst_v], add=True)
            return carry

        lax.fori_loop(0, n_chunks, chunk, 0)

        plsc.subcore_barrier()

        @pl.when(sid == 0)
        def _():
            pltpu.sync_copy(acc, out_h.at[pl.ds(cid * N, N)])

    return sc_edge


_sc_l01 = _make_sc_edge(False)
_sc_l2 = _make_sc_edge(True)


def _fold(Wx, av):
    hn, d = av.shape
    return jnp.einsum("khj,hj->kh", Wx.reshape(Wx.shape[0], hn, d), av)


def kernel(x, edge_index, Ws0, Wd0, al0, ar0, b0, Ws1, Wd1, al1, ar1, b1,
           Ws2, Wd2, al2, ar2, b2):
    src = edge_index[0]
    dst = edge_index[1]

    wcat0 = jnp.concatenate([Ws0, _fold(Ws0, al0), _fold(Wd0, ar0)], axis=1)
    wcat1 = jnp.concatenate([Ws1, _fold(Ws1, al1), _fold(Wd1, ar1)], axis=1)
    wcat2 = jnp.concatenate([Ws2, _fold(Ws2, al2), _fold(Wd2, ar2)], axis=1)

    cols64 = np.arange(64)
    b8 = jnp.asarray((cols64 // 8 == np.arange(8)[:, None]).astype(np.float32))
    cols80 = np.arange(80)
    bc2 = jnp.asarray((cols80 // 40 == np.arange(2)[:, None]).astype(np.float32))
    s80 = jnp.asarray((cols80[:, None] % 40 == np.arange(40)[None, :]
                       ).astype(np.float32))
    bbar = b2.reshape(H, C).mean(axis=0).reshape(1, C)
    z128 = jnp.zeros((N, 128), jnp.float32)

    a0, er0 = _tc_prep0(x, wcat0)
    acc0 = _sc_l01(src, dst, a0, er0, z128).reshape(2, N, 128)
    a1, er1 = _tc_mid(acc0, b0.reshape(1, 64), wcat1, b8)
    acc1 = _sc_l01(src, dst, a1, er1, z128).reshape(2, N, 128)
    a2a, a2b, e2a, e2b = _tc_prep2(acc1, b1.reshape(1, 64), wcat2, b8)
    acc2a = _sc_l2(src, dst, a2a.reshape(2 * N, 128),
                   e2a.reshape(2 * N, 128), z128).reshape(2, N, 128)
    acc2b = _sc_l2(src, dst, a2b.reshape(2 * N, 128),
                   e2b.reshape(2 * N, 128), z128).reshape(2, N, 128)
    return _tc_final(acc2a, acc2b, bc2, s80, bbar)
